# Initial kernel scaffold; baseline (speedup 1.0000x reference)
import jax
import jax.numpy as jnp
from jax.experimental import pallas as pl


def kernel(x, freq_embed, time_embed):
    raise NotImplementedError("write your pallas kernel here")


# speedup vs baseline: 1.5274x; 1.5274x over previous
import jax
import jax.numpy as jnp
from jax.experimental import pallas as pl

MAX_F, MAX_T = 64, 512
N_EMBD = 512
N_EMBD_F = 256
N_EMBD_T = 256

_FB = 8


def _add_pos_kernel(x_ref, f_ref, t_ref, o_ref):
    xb = x_ref[0]
    fb = f_ref[...]
    tb = t_ref[...]
    o_ref[0, :, :, :N_EMBD_F] = xb[:, :, :N_EMBD_F] + fb[:, None, :]
    o_ref[0, :, :, N_EMBD_F:] = xb[:, :, N_EMBD_F:] + tb[None, :, :]


def kernel(x, freq_embed, time_embed):
    B, F, T, D = x.shape
    grid = (B, F // _FB)
    return pl.pallas_call(
        _add_pos_kernel,
        grid=grid,
        in_specs=[
            pl.BlockSpec((1, _FB, T, D), lambda b, f: (b, f, 0, 0)),
            pl.BlockSpec((_FB, N_EMBD_F), lambda b, f: (f, 0)),
            pl.BlockSpec((T, N_EMBD_T), lambda b, f: (0, 0)),
        ],
        out_specs=pl.BlockSpec((1, _FB, T, D), lambda b, f: (b, f, 0, 0)),
        out_shape=jax.ShapeDtypeStruct(x.shape, x.dtype),
    )(x, freq_embed, time_embed)
